# Initial kernel scaffold; baseline (speedup 1.0000x reference)
import jax
import jax.numpy as jnp
from jax.experimental import pallas as pl


def kernel(x, W, b, connectivity_values, connectivity_indices):
    raise NotImplementedError("write your pallas kernel here")


# speedup vs baseline: 34.9634x; 34.9634x over previous
import functools

import numpy as np
import jax
import jax.numpy as jnp
from jax import lax
from jax.experimental import pallas as pl
from jax.experimental.pallas import tpu as pltpu
from jax.experimental.pallas import tpu_sc as plsc

_POOL = 16384
_DIM = 1024
_BATCH = 4
_SPARSITY = 0.001
_NNZ = int(_POOL * _POOL * _SPARSITY)

_NTOT = _BATCH * _POOL
_QN = np.float32(1.0 - _SPARSITY) * (np.float32(_NTOT) - np.float32(1.0))
_LOW = int(np.floor(_QN))
_HIGH = int(np.ceil(_QN))
_HW = np.float32(_QN - np.float32(_LOW))
_LW = np.float32(1.0) - _HW
_K_LO = _NTOT - _LOW
_K_HI = _NTOT - _HIGH

_NC = 2
_NS = 16
_NW = _NC * _NS
_EPW = 8400
_PADN = _EPW * _NW

_TILES = 16
_TP = _POOL // _TILES


def _mm_sel_body(x_ref, w_ref, b_ref, state_ref, act_s):
    i = pl.program_id(0)
    acc = lax.dot_general(
        x_ref[...], w_ref[...], (((1,), (1,)), ((), ())),
        preferred_element_type=jnp.float32,
    )
    act_s[:, pl.ds(i * _TP, _TP)] = acc + b_ref[...][None, :]

    @pl.when(i == _TILES - 1)
    def _select():
        bits = lax.bitcast_convert_type(jnp.abs(act_s[...]), jnp.int32)

        def step(j, carry):
            p67, p66 = carry
            bit = jnp.int32(1) << (jnp.int32(30) - j)
            c67 = p67 + bit
            c66 = p66 + bit
            n67 = jnp.sum((bits >= c67).astype(jnp.int32))
            n66 = jnp.sum((bits >= c66).astype(jnp.int32))
            p67 = jnp.where(n67 >= _K_LO, c67, p67)
            p66 = jnp.where(n66 >= _K_HI, c66, p66)
            return (p67, p66)

        p67, p66 = lax.fori_loop(0, 31, step, (jnp.int32(0), jnp.int32(0)))
        v_lo = lax.bitcast_convert_type(jnp.full((8, 128), p67, jnp.int32),
                                        jnp.float32)
        v_hi = lax.bitcast_convert_type(jnp.full((8, 128), p66, jnp.int32),
                                        jnp.float32)
        thr = jnp.max(v_lo * _LW + v_hi * _HW)
        a0 = act_s[0:1, :]
        state_ref[...] = jnp.where(jnp.abs(a0) > thr, a0, jnp.float32(0.0))


def _matmul_select(x, W, b):
    return pl.pallas_call(
        _mm_sel_body,
        grid=(_TILES,),
        in_specs=[
            pl.BlockSpec((_BATCH, _DIM), lambda i: (0, 0)),
            pl.BlockSpec((_TP, _DIM), lambda i: (i, 0)),
            pl.BlockSpec((_TP,), lambda i: (i,)),
        ],
        out_specs=pl.BlockSpec((1, _POOL), lambda i: (0, 0)),
        out_shape=jax.ShapeDtypeStruct((1, _POOL), jnp.float32),
        scratch_shapes=[pltpu.VMEM((_BATCH, _POOL), jnp.float32)],
    )(x, W, b)


def _sc_body(state_hbm, rows_hbm, cols_hbm, vals_hbm, out_hbm,
             rows_v, cols_v, vals_v, state_v, acc_v, sem):
    c = lax.axis_index("c")
    s = lax.axis_index("s")
    wid = s * _NC + c
    base = wid * _EPW
    cp0 = pltpu.async_copy(state_hbm, state_v, sem)
    cp1 = pltpu.async_copy(rows_hbm.at[pl.ds(base, _EPW)], rows_v, sem)
    cp2 = pltpu.async_copy(cols_hbm.at[pl.ds(base, _EPW)], cols_v, sem)
    cp3 = pltpu.async_copy(vals_hbm.at[pl.ds(base, _EPW)], vals_v, sem)

    def zstep(i, carry):
        acc_v[pl.ds(i * 16, 16)] = jnp.zeros((16,), jnp.float32)
        return carry

    lax.fori_loop(0, _POOL // 16, zstep, 0, unroll=8)
    cp0.wait()
    cp1.wait()
    cp2.wait()
    cp3.wait()

    def estep(i, carry):
        idx = cols_v[pl.ds(i * 16, 16)]
        g = plsc.load_gather(state_v, [idx])
        contrib = vals_v[pl.ds(i * 16, 16)] * g
        plsc.addupdate_scatter(acc_v, [rows_v[pl.ds(i * 16, 16)]], contrib)
        return carry

    lax.fori_loop(0, _EPW // 16, estep, 0, unroll=8)

    pltpu.sync_copy(acc_v, out_hbm.at[wid])


def _sc_spmv(state, rows, cols, vals):
    mesh = plsc.VectorSubcoreMesh(core_axis_name="c", subcore_axis_name="s")
    f = pl.kernel(
        _sc_body,
        out_type=jax.ShapeDtypeStruct((_NW, _POOL), jnp.float32),
        mesh=mesh,
        compiler_params=pltpu.CompilerParams(needs_layout_passes=False),
        scratch_types=[
            pltpu.VMEM((_EPW,), jnp.int32),
            pltpu.VMEM((_EPW,), jnp.int32),
            pltpu.VMEM((_EPW,), jnp.float32),
            pltpu.VMEM((_POOL,), jnp.float32),
            pltpu.VMEM((_POOL,), jnp.float32),
            pltpu.SemaphoreType.DMA,
        ],
    )
    return f(state, rows, cols, vals)


def _fin_body(state_ref, part_ref, out_ref):
    r = jnp.sum(part_ref[...], axis=0, keepdims=True)
    xv = state_ref[...] + jnp.float32(0.1) * r
    out_ref[...] = jnp.float32(0.5) * xv * (
        jnp.float32(1.0) + lax.erf(xv * np.float32(np.sqrt(0.5))))


def _finalize(state, partials):
    return pl.pallas_call(
        _fin_body,
        out_shape=jax.ShapeDtypeStruct((1, _POOL), jnp.float32),
    )(state, partials)


def kernel(x, W, b, connectivity_values, connectivity_indices):
    state = _matmul_select(x, W, b)

    rows = connectivity_indices[0]
    cols = connectivity_indices[1]
    pad = _PADN - _NNZ
    rows_p = jnp.concatenate([rows, jnp.zeros((pad,), jnp.int32)])
    cols_p = jnp.concatenate([cols, jnp.zeros((pad,), jnp.int32)])
    vals_p = jnp.concatenate(
        [connectivity_values, jnp.zeros((pad,), jnp.float32)])

    state1 = state.reshape(_POOL)
    partials = _sc_spmv(state1, rows_p, cols_p, vals_p)
    out = _finalize(state, partials)
    return out.reshape(_POOL)


# speedup vs baseline: 37.1088x; 1.0614x over previous
import functools

import numpy as np
import jax
import jax.numpy as jnp
from jax import lax
from jax.experimental import pallas as pl
from jax.experimental.pallas import tpu as pltpu
from jax.experimental.pallas import tpu_sc as plsc

_POOL = 16384
_DIM = 1024
_BATCH = 4
_SPARSITY = 0.001
_NNZ = int(_POOL * _POOL * _SPARSITY)

_NTOT = _BATCH * _POOL
_QN = np.float32(1.0 - _SPARSITY) * (np.float32(_NTOT) - np.float32(1.0))
_LOW = int(np.floor(_QN))
_HIGH = int(np.ceil(_QN))
_HW = np.float32(_QN - np.float32(_LOW))
_LW = np.float32(1.0) - _HW
_K_LO = _NTOT - _LOW
_K_HI = _NTOT - _HIGH

_NC = 2
_NS = 16
_NW = _NC * _NS
_EPW = 8384
_W31_BASE = _EPW * (_NW - 1)
_W31_MAIN = (_NNZ - _W31_BASE) // 16 * 16
_TAIL_BASE = _W31_BASE + _W31_MAIN
_TAIL_N = _NNZ - _TAIL_BASE
_BUF = _W31_MAIN + 16
_ITERS = _BUF // 16

_TILES = 16
_TP = _POOL // _TILES
_HP = _POOL // 2


def _mm_sel_body(x_ref, w_ref, b_ref, state_ref, bits_s, a0_s):
    i = pl.program_id(0)
    acc = lax.dot_general(
        x_ref[...], w_ref[...], (((1,), (1,)), ((), ())),
        preferred_element_type=jnp.float32,
    )
    t = acc + b_ref[...][None, :]
    a0_s[:, pl.ds(i * _TP, _TP)] = t[0:1, :]
    t8 = jnp.concatenate([t[:, :_TP // 2], t[:, _TP // 2:]], axis=0)
    bits_s[:, pl.ds(i * (_TP // 2), _TP // 2)] = (
        lax.bitcast_convert_type(jnp.abs(t8), jnp.int32))

    @pl.when(i == _TILES - 1)
    def _select():
        bits = bits_s[...]

        def step(j, p66):
            c66 = p66 + (jnp.int32(1) << (jnp.int32(30) - j))
            n66 = jnp.sum((bits >= c66).astype(jnp.int32))
            return jnp.where(n66 >= _K_HI, c66, p66)

        p66 = lax.fori_loop(0, 31, step, jnp.int32(0))
        bits2 = bits_s[...]
        n_ge = jnp.sum((bits2 >= p66).astype(jnp.int32))
        below_max = jnp.max(jnp.where(bits2 < p66, bits2, jnp.int32(0)))
        p67 = jnp.where(n_ge >= _K_LO, p66, below_max)
        v_lo = lax.bitcast_convert_type(jnp.full((8, 128), p67, jnp.int32),
                                        jnp.float32)
        v_hi = lax.bitcast_convert_type(jnp.full((8, 128), p66, jnp.int32),
                                        jnp.float32)
        thr = jnp.max(v_lo * _LW + v_hi * _HW)
        a0 = a0_s[...]
        state_ref[...] = jnp.where(jnp.abs(a0) > thr, a0, jnp.float32(0.0))


def _matmul_select(x, W, b):
    return pl.pallas_call(
        _mm_sel_body,
        grid=(_TILES,),
        in_specs=[
            pl.BlockSpec((_BATCH, _DIM), lambda i: (0, 0)),
            pl.BlockSpec((_TP, _DIM), lambda i: (i, 0)),
            pl.BlockSpec((_TP,), lambda i: (i,)),
        ],
        out_specs=pl.BlockSpec((1, _POOL), lambda i: (0, 0)),
        out_shape=jax.ShapeDtypeStruct((1, _POOL), jnp.float32),
        scratch_shapes=[
            pltpu.VMEM((8, _HP), jnp.int32),
            pltpu.VMEM((1, _POOL), jnp.float32),
        ],
    )(x, W, b)


def _sc_body(state_hbm, rows_hbm, cols_hbm, vals_hbm,
             trow_hbm, tcol_hbm, tval_hbm, out_hbm,
             rows_v, cols_v, vals_v, state_v, acc_v, sem):
    c = lax.axis_index("c")
    s = lax.axis_index("s")
    wid = s * _NC + c
    is31 = wid == _NW - 1
    base = wid * _EPW
    cp0 = pltpu.async_copy(state_hbm, state_v, sem)
    cp1 = pltpu.async_copy(rows_hbm.at[pl.ds(base, _EPW)], rows_v.at[pl.ds(0, _EPW)], sem)
    cp2 = pltpu.async_copy(cols_hbm.at[pl.ds(base, _EPW)], cols_v.at[pl.ds(0, _EPW)], sem)
    cp3 = pltpu.async_copy(vals_hbm.at[pl.ds(base, _EPW)], vals_v.at[pl.ds(0, _EPW)], sem)

    zeros16i = jnp.zeros((16,), jnp.int32)
    zeros16f = jnp.zeros((16,), jnp.float32)
    for z in range(_EPW, _BUF, 16):
        rows_v[pl.ds(z, 16)] = zeros16i
        cols_v[pl.ds(z, 16)] = zeros16i
        vals_v[pl.ds(z, 16)] = zeros16f

    _EXTRA = _W31_MAIN - _EPW

    @pl.when(is31)
    def _tail():
        pltpu.sync_copy(rows_hbm.at[pl.ds(_W31_BASE + _EPW, _EXTRA)],
                        rows_v.at[pl.ds(_EPW, _EXTRA)])
        pltpu.sync_copy(cols_hbm.at[pl.ds(_W31_BASE + _EPW, _EXTRA)],
                        cols_v.at[pl.ds(_EPW, _EXTRA)])
        pltpu.sync_copy(vals_hbm.at[pl.ds(_W31_BASE + _EPW, _EXTRA)],
                        vals_v.at[pl.ds(_EPW, _EXTRA)])
        pltpu.sync_copy(trow_hbm, rows_v.at[pl.ds(_W31_MAIN, 16)])
        pltpu.sync_copy(tcol_hbm, cols_v.at[pl.ds(_W31_MAIN, 16)])
        pltpu.sync_copy(tval_hbm, vals_v.at[pl.ds(_W31_MAIN, 16)])

    def zstep(i, carry):
        acc_v[pl.ds(i * 16, 16)] = jnp.zeros((16,), jnp.float32)
        return carry

    lax.fori_loop(0, _POOL // 16, zstep, 0, unroll=8)
    cp0.wait()
    cp1.wait()
    cp2.wait()
    cp3.wait()

    def estep(i, carry):
        idx = cols_v[pl.ds(i * 16, 16)]
        g = plsc.load_gather(state_v, [idx])
        contrib = vals_v[pl.ds(i * 16, 16)] * g
        plsc.addupdate_scatter(acc_v, [rows_v[pl.ds(i * 16, 16)]], contrib)
        return carry

    lax.fori_loop(0, _ITERS, estep, 0, unroll=8)

    pltpu.sync_copy(acc_v, out_hbm.at[wid])


def _sc_spmv(state, rows, cols, vals, trow, tcol, tval):
    mesh = plsc.VectorSubcoreMesh(core_axis_name="c", subcore_axis_name="s")
    f = pl.kernel(
        _sc_body,
        out_type=jax.ShapeDtypeStruct((_NW, _POOL), jnp.float32),
        mesh=mesh,
        compiler_params=pltpu.CompilerParams(needs_layout_passes=False),
        scratch_types=[
            pltpu.VMEM((_BUF,), jnp.int32),
            pltpu.VMEM((_BUF,), jnp.int32),
            pltpu.VMEM((_BUF,), jnp.float32),
            pltpu.VMEM((_POOL,), jnp.float32),
            pltpu.VMEM((_POOL,), jnp.float32),
            pltpu.SemaphoreType.DMA,
        ],
    )
    return f(state, rows, cols, vals, trow, tcol, tval)


def _fin_body(state_ref, part_ref, out_ref):
    r = jnp.sum(part_ref[...], axis=0, keepdims=True)
    xv = state_ref[...] + jnp.float32(0.1) * r
    out_ref[...] = jnp.float32(0.5) * xv * (
        jnp.float32(1.0) + lax.erf(xv * np.float32(np.sqrt(0.5))))


def _finalize(state, partials):
    return pl.pallas_call(
        _fin_body,
        out_shape=jax.ShapeDtypeStruct((1, _POOL), jnp.float32),
    )(state, partials)


def kernel(x, W, b, connectivity_values, connectivity_indices):
    state = _matmul_select(x, W, b)

    rows = connectivity_indices[0]
    cols = connectivity_indices[1]
    zpad = jnp.zeros((16 - _TAIL_N,), jnp.int32)
    trow = jnp.concatenate([lax.slice(rows, [_TAIL_BASE], [_NNZ]), zpad])
    tcol = jnp.concatenate([lax.slice(cols, [_TAIL_BASE], [_NNZ]), zpad])
    tval = jnp.concatenate(
        [lax.slice(connectivity_values, [_TAIL_BASE], [_NNZ]),
         jnp.zeros((16 - _TAIL_N,), jnp.float32)])

    state1 = state.reshape(_POOL)
    partials = _sc_spmv(state1, rows, cols, connectivity_values,
                        trow, tcol, tval)
    out = _finalize(state, partials)
    return out.reshape(_POOL)


# speedup vs baseline: 38.2474x; 1.0307x over previous
import functools

import numpy as np
import jax
import jax.numpy as jnp
from jax import lax
from jax.experimental import pallas as pl
from jax.experimental.pallas import tpu as pltpu
from jax.experimental.pallas import tpu_sc as plsc

_POOL = 16384
_DIM = 1024
_BATCH = 4
_SPARSITY = 0.001
_NNZ = int(_POOL * _POOL * _SPARSITY)

_NTOT = _BATCH * _POOL
_QN = np.float32(1.0 - _SPARSITY) * (np.float32(_NTOT) - np.float32(1.0))
_LOW = int(np.floor(_QN))
_HIGH = int(np.ceil(_QN))
_HW = np.float32(_QN - np.float32(_LOW))
_LW = np.float32(1.0) - _HW
_K_LO = _NTOT - _LOW
_K_HI = _NTOT - _HIGH

_NC = 2
_NS = 16
_NW = _NC * _NS
_EPW = 8384
_W31_BASE = _EPW * (_NW - 1)
_W31_MAIN = (_NNZ - _W31_BASE) // 16 * 16
_TAIL_BASE = _W31_BASE + _W31_MAIN
_TAIL_N = _NNZ - _TAIL_BASE
_BUF = _W31_MAIN + 16
_ITERS = _BUF // 16

_TILES = 8
_TP = _POOL // _TILES
_HP = _POOL // 2


def _mm_sel_body(x_ref, w_ref, b_ref, state_ref, bits_s, a0_s):
    i = pl.program_id(0)
    acc = lax.dot_general(
        x_ref[...], w_ref[...], (((1,), (1,)), ((), ())),
        preferred_element_type=jnp.float32,
    )
    t = acc + b_ref[...][None, :]
    a0_s[:, pl.ds(i * _TP, _TP)] = t[0:1, :]
    t8 = jnp.concatenate([t[:, :_TP // 2], t[:, _TP // 2:]], axis=0)
    bits_s[:, pl.ds(i * (_TP // 2), _TP // 2)] = (
        lax.bitcast_convert_type(jnp.abs(t8), jnp.int32))

    @pl.when(i == _TILES - 1)
    def _select():
        bits = bits_s[...]

        def step(j, p66):
            c66 = p66 + (jnp.int32(1) << (jnp.int32(30) - j))
            n66 = jnp.sum((bits >= c66).astype(jnp.int32))
            return jnp.where(n66 >= _K_HI, c66, p66)

        p66 = lax.fori_loop(0, 31, step, jnp.int32(0))
        bits2 = bits_s[...]
        n_ge = jnp.sum((bits2 >= p66).astype(jnp.int32))
        below_max = jnp.max(jnp.where(bits2 < p66, bits2, jnp.int32(0)))
        p67 = jnp.where(n_ge >= _K_LO, p66, below_max)
        v_lo = lax.bitcast_convert_type(jnp.full((8, 128), p67, jnp.int32),
                                        jnp.float32)
        v_hi = lax.bitcast_convert_type(jnp.full((8, 128), p66, jnp.int32),
                                        jnp.float32)
        thr = jnp.max(v_lo * _LW + v_hi * _HW)
        a0 = a0_s[...]
        state_ref[...] = jnp.where(jnp.abs(a0) > thr, a0, jnp.float32(0.0))


def _matmul_select(x, W, b):
    return pl.pallas_call(
        _mm_sel_body,
        grid=(_TILES,),
        in_specs=[
            pl.BlockSpec((_BATCH, _DIM), lambda i: (0, 0)),
            pl.BlockSpec((_TP, _DIM), lambda i: (i, 0)),
            pl.BlockSpec((_TP,), lambda i: (i,)),
        ],
        out_specs=pl.BlockSpec((1, _POOL), lambda i: (0, 0)),
        out_shape=jax.ShapeDtypeStruct((1, _POOL), jnp.float32),
        scratch_shapes=[
            pltpu.VMEM((8, _HP), jnp.int32),
            pltpu.VMEM((1, _POOL), jnp.float32),
        ],
    )(x, W, b)


def _sc_body(state_hbm, rows_hbm, cols_hbm, vals_hbm,
             trow_hbm, tcol_hbm, tval_hbm, out_hbm,
             rows_v, cols_v, vals_v, state_v, acc_v, sem):
    c = lax.axis_index("c")
    s = lax.axis_index("s")
    wid = s * _NC + c
    is31 = wid == _NW - 1
    base = wid * _EPW
    cp0 = pltpu.async_copy(state_hbm, state_v, sem)
    cp1 = pltpu.async_copy(rows_hbm.at[pl.ds(base, _EPW)], rows_v.at[pl.ds(0, _EPW)], sem)
    cp2 = pltpu.async_copy(cols_hbm.at[pl.ds(base, _EPW)], cols_v.at[pl.ds(0, _EPW)], sem)
    cp3 = pltpu.async_copy(vals_hbm.at[pl.ds(base, _EPW)], vals_v.at[pl.ds(0, _EPW)], sem)

    zeros16i = jnp.zeros((16,), jnp.int32)
    zeros16f = jnp.zeros((16,), jnp.float32)
    for z in range(_EPW, _BUF, 16):
        rows_v[pl.ds(z, 16)] = zeros16i
        cols_v[pl.ds(z, 16)] = zeros16i
        vals_v[pl.ds(z, 16)] = zeros16f

    _EXTRA = _W31_MAIN - _EPW

    @pl.when(is31)
    def _tail():
        pltpu.sync_copy(rows_hbm.at[pl.ds(_W31_BASE + _EPW, _EXTRA)],
                        rows_v.at[pl.ds(_EPW, _EXTRA)])
        pltpu.sync_copy(cols_hbm.at[pl.ds(_W31_BASE + _EPW, _EXTRA)],
                        cols_v.at[pl.ds(_EPW, _EXTRA)])
        pltpu.sync_copy(vals_hbm.at[pl.ds(_W31_BASE + _EPW, _EXTRA)],
                        vals_v.at[pl.ds(_EPW, _EXTRA)])
        pltpu.sync_copy(trow_hbm, rows_v.at[pl.ds(_W31_MAIN, 16)])
        pltpu.sync_copy(tcol_hbm, cols_v.at[pl.ds(_W31_MAIN, 16)])
        pltpu.sync_copy(tval_hbm, vals_v.at[pl.ds(_W31_MAIN, 16)])

    def zstep(i, carry):
        acc_v[pl.ds(i * 16, 16)] = jnp.zeros((16,), jnp.float32)
        return carry

    lax.fori_loop(0, _POOL // 16, zstep, 0, unroll=8)
    cp0.wait()
    cp1.wait()
    cp2.wait()
    cp3.wait()

    def estep(i, carry):
        idx = cols_v[pl.ds(i * 16, 16)]
        g = plsc.load_gather(state_v, [idx])
        contrib = vals_v[pl.ds(i * 16, 16)] * g
        plsc.addupdate_scatter(acc_v, [rows_v[pl.ds(i * 16, 16)]], contrib)
        return carry

    lax.fori_loop(0, _ITERS, estep, 0, unroll=8)

    pltpu.sync_copy(acc_v, out_hbm.at[wid])


def _sc_spmv(state, rows, cols, vals, trow, tcol, tval):
    mesh = plsc.VectorSubcoreMesh(core_axis_name="c", subcore_axis_name="s")
    f = pl.kernel(
        _sc_body,
        out_type=jax.ShapeDtypeStruct((_NW, _POOL), jnp.float32),
        mesh=mesh,
        compiler_params=pltpu.CompilerParams(needs_layout_passes=False),
        scratch_types=[
            pltpu.VMEM((_BUF,), jnp.int32),
            pltpu.VMEM((_BUF,), jnp.int32),
            pltpu.VMEM((_BUF,), jnp.float32),
            pltpu.VMEM((_POOL,), jnp.float32),
            pltpu.VMEM((_POOL,), jnp.float32),
            pltpu.SemaphoreType.DMA,
        ],
    )
    return f(state, rows, cols, vals, trow, tcol, tval)


def _fin_body(state_ref, part_ref, out_ref):
    r = jnp.sum(part_ref[...], axis=0, keepdims=True)
    xv = state_ref[...] + jnp.float32(0.1) * r
    out_ref[...] = jnp.float32(0.5) * xv * (
        jnp.float32(1.0) + lax.erf(xv * np.float32(np.sqrt(0.5))))


def _finalize(state, partials):
    return pl.pallas_call(
        _fin_body,
        out_shape=jax.ShapeDtypeStruct((1, _POOL), jnp.float32),
    )(state, partials)


def kernel(x, W, b, connectivity_values, connectivity_indices):
    state = _matmul_select(x, W, b)

    rows = connectivity_indices[0]
    cols = connectivity_indices[1]
    zpad = jnp.zeros((16 - _TAIL_N,), jnp.int32)
    trow = jnp.concatenate([lax.slice(rows, [_TAIL_BASE], [_NNZ]), zpad])
    tcol = jnp.concatenate([lax.slice(cols, [_TAIL_BASE], [_NNZ]), zpad])
    tval = jnp.concatenate(
        [lax.slice(connectivity_values, [_TAIL_BASE], [_NNZ]),
         jnp.zeros((16 - _TAIL_N,), jnp.float32)])

    state1 = state.reshape(_POOL)
    partials = _sc_spmv(state1, rows, cols, connectivity_values,
                        trow, tcol, tval)
    out = _finalize(state, partials)
    return out.reshape(_POOL)
